# baseline (speedup 1.0000x reference)
import jax
import jax.numpy as jnp
import numpy as np
from jax import lax
from jax.experimental import pallas as pl
from jax.experimental.pallas import tpu as pltpu
from jax.experimental.pallas import tpu_sc as plsc

T = 16384
D = 128
H = 16
G = 8
NUM_EMB = 100000
OE_M = 100003
OE_ROWS = 800088
OE_LINES = 100016

NC = 2
NS = 16
NW = NC * NS
TPW = T // NW
HALF = TPW // 2
PAD = 8

CH = 2048
CHP = CH + 1
NCH_FULL = 390
NJ = 13
TAIL1_C0 = NCH_FULL * CH
TAIL1_W = 800000 - TAIL1_C0
TAIL2_C0 = 800000

_EXCL = [0] * (G + 1)
for _i in range(G):
    _EXCL[_i + 1] = _EXCL[_i] + OE_M + 2 * _i + 1


def _fmod(x, m, inv):
    m = np.int32(m)
    q = (x.astype(jnp.float32) * inv).astype(jnp.int32)
    r = x - q * m
    r = jnp.where(r < np.int32(0), r + m, r)
    r = jnp.where(r >= m, r - m, r)
    return r


def _tr_body(oet_hbm, tail_hbm, lines_out, chunk_v, out_v, isem0, isem1,
             osem):
    cid = lax.axis_index("c")
    sid = lax.axis_index("s")
    wid = sid * np.int32(NC) + cid
    iota16 = lax.iota(jnp.int32, 16)
    zeros16 = jnp.full((16,), 0, jnp.int32)
    scol = [iota16 + np.int32(16 * k) for k in range(8)]
    isems = [isem0, isem1]

    def in_copy2(jj, b):
        ch = wid + jj * np.int32(NW)
        return pltpu.make_async_copy(
            oet_hbm.at[:, pl.ds(ch * np.int32(CH), CH)],
            chunk_v.at[b, :, pl.ds(0, CH)], isems[b])

    def extract(src2d, L):
        @plsc.parallel_loop(np.int32(0), np.int32(L), np.int32(1), unroll=8)
        def lbody(l):
            base8 = l * np.int32(8)
            lrow = zeros16 + l
            for k in range(8):
                col = zeros16 + (base8 + np.int32(k))
                vals = plsc.load_gather(src2d, [iota16, col])
                plsc.store_scatter(out_v, [lrow, scol[k]], vals)

    def out_copy(lbase, L):
        return pltpu.make_async_copy(
            out_v.at[pl.ds(0, L)], lines_out.at[pl.ds(lbase, L)], osem)

    @pl.when(wid < np.int32(NCH_FULL))
    def _():
        in_copy2(np.int32(0), 0).start()

    def j2body(j2, _):
        for b in range(2):
            jj = j2 * np.int32(2) + np.int32(b)
            ch = wid + jj * np.int32(NW)
            chn = ch + np.int32(NW)

            @pl.when(chn < np.int32(NCH_FULL))
            def _(jj=jj, b=b):
                in_copy2(jj + np.int32(1), 1 - b).start()

            @pl.when(ch < np.int32(NCH_FULL))
            def _(jj=jj, ch=ch, b=b):
                in_copy2(jj, b).wait()

                @pl.when(jj >= np.int32(1))
                def _():
                    out_copy(np.int32(0), 256).wait()

                extract(chunk_v.at[b], 256)
                out_copy(ch * np.int32(256), 256).start()

        return np.int32(0)

    lax.fori_loop(np.int32(0), np.int32((NJ + 1) // 2), j2body, np.int32(0))

    out_copy(np.int32(0), 256).wait()

    @pl.when(wid == np.int32(0))
    def _():
        pltpu.sync_copy(oet_hbm.at[:, pl.ds(TAIL1_C0, TAIL1_W)],
                        chunk_v.at[0, :, pl.ds(0, TAIL1_W)])
        extract(chunk_v.at[0], TAIL1_W // 8)
        out_copy(np.int32(TAIL1_C0 // 8), TAIL1_W // 8).start()
        out_copy(np.int32(0), TAIL1_W // 8).wait()

    @pl.when(wid == np.int32(1))
    def _():
        pltpu.sync_copy(tail_hbm, chunk_v.at[0, :, pl.ds(0, 128)])
        extract(chunk_v.at[0], 16)
        out_copy(np.int32(TAIL2_C0 // 8), 16).start()
        out_copy(np.int32(0), 16).wait()


_sc_transpose = pl.kernel(
    _tr_body,
    out_type=jax.ShapeDtypeStruct((OE_LINES, D), jnp.float32),
    mesh=plsc.VectorSubcoreMesh(
        core_axis_name="c", subcore_axis_name="s",
        num_cores=NC, num_subcores=NS),
    scratch_types=[
        pltpu.VMEM((2, 16, CHP), jnp.float32),
        pltpu.VMEM((256, D), jnp.float32),
        pltpu.SemaphoreType.DMA,
        pltpu.SemaphoreType.DMA,
        pltpu.SemaphoreType.DMA,
    ],
    compiler_params=pltpu.CompilerParams(needs_layout_passes=False),
)


def _sc_body(padded_hbm, wt_hbm, oel_hbm, word_out, hid_out,
             toks_v, lidx_v, soff_v, widx_v, lines_v, hid_v, wrows_v,
             lsem0, lsem1, wsem):
    cid = lax.axis_index("c")
    sid = lax.axis_index("s")
    wid = sid * np.int32(NC) + cid
    base = wid * np.int32(TPW)
    iota16 = lax.iota(jnp.int32, 16)
    rot = [(iota16 + np.int32(h)) & np.int32(15) for h in range(16)]
    lsems = [lsem0, lsem1]

    def line_copy(g, s, b):
        return pltpu.make_async_copy(
            oel_hbm.at[lidx_v.at[2 * g + s]], lines_v.at[b], lsems[b])

    def half_body(half, _):
        hbase = base + half * np.int32(HALF)
        pltpu.sync_copy(padded_hbm.at[pl.ds(hbase, HALF + PAD)], toks_v)

        for s in range(2):
            wrow = widx_v.at[s]

            def wbody(i, _, s=s, wrow=wrow):
                off = i * np.int32(16) + np.int32(s * 128 + PAD)
                wrow[pl.ds(i * np.int32(16), 16)] = toks_v[pl.ds(off, 16)]
                return np.int32(0)

            lax.fori_loop(np.int32(0), np.int32(8), wbody, np.int32(0))
        wdescs = [
            pltpu.make_async_copy(
                wt_hbm.at[widx_v.at[s]],
                wrows_v.at[pl.ds(s * 128, 128)], wsem)
            for s in range(2)
        ]
        for dsc in wdescs:
            dsc.start()

        for g in range(G):
            m = 100004 + 2 * g
            c = m - NUM_EMB
            inv = np.float32(1.0 / m)
            excl = _EXCL[g]
            for s in range(2):
                lrow = lidx_v.at[2 * g + s]
                orow = soff_v.at[2 * g + s]

                def gbody(i, _, s=s, m=m, c=c, inv=inv, excl=excl,
                          lrow=lrow, orow=orow, is3=(g >= 4)):
                    off = i * np.int32(16) + np.int32(s * 128 + PAD)
                    t0 = toks_v[pl.ds(off, 16)]
                    t1 = toks_v[pl.ds(off - np.int32(1), 16)]
                    r1 = _fmod(np.int32(c) * t1, m, inv)
                    acc = t0 - r1 + np.int32(m)
                    if is3:
                        t2 = toks_v[pl.ds(off - np.int32(2), 16)]
                        rr2 = _fmod(
                            np.int32(c) * _fmod(np.int32(c) * t2, m, inv),
                            m, inv)
                        acc = acc + rr2
                    gid = _fmod(acc, m, inv) + np.int32(excl)
                    j = i * np.int32(16)
                    lrow[pl.ds(j, 16)] = lax.shift_right_logical(
                        gid, np.int32(3))
                    orow[pl.ds(j, 16)] = lax.shift_left(
                        gid & np.int32(7), np.int32(4))
                    return np.int32(0)

                lax.fori_loop(np.int32(0), np.int32(8), gbody, np.int32(0))

        pairs = [(g, s) for g in range(G) for s in range(2)]
        line_copy(0, 0, 0).start()
        for p, (g, s) in enumerate(pairs):
            b = p % 2
            if p + 1 < len(pairs):
                gn, sn = pairs[p + 1]
                line_copy(gn, sn, 1 - b).start()
            line_copy(g, s, b).wait()
            src = lines_v.at[b]
            orow = soff_v.at[2 * g + s]

            @plsc.parallel_loop(np.int32(0), np.int32(8), np.int32(1),
                                unroll=4)
            def ebody(j, g=g, s=s, src=src, orow=orow):
                soff16 = orow[pl.ds(j * np.int32(16), 16)]
                lrows = iota16 + j * np.int32(16)
                trows = lrows + np.int32(s * 128)
                for h in range(16):
                    vals = plsc.load_gather(src, [lrows, soff16 + rot[h]])
                    plsc.store_scatter(
                        hid_v, [trows, rot[h] + np.int32(16 * g)], vals)

        for dsc in wdescs:
            dsc.wait()
        pltpu.sync_copy(wrows_v, word_out.at[pl.ds(hbase, HALF)])
        pltpu.sync_copy(hid_v, hid_out.at[pl.ds(hbase, HALF)])
        return np.int32(0)

    lax.fori_loop(np.int32(0), np.int32(2), half_body, np.int32(0))


_sc_gather = pl.kernel(
    _sc_body,
    out_type=(
        jax.ShapeDtypeStruct((T, D), jnp.float32),
        jax.ShapeDtypeStruct((T, D), jnp.float32),
    ),
    mesh=plsc.VectorSubcoreMesh(
        core_axis_name="c", subcore_axis_name="s",
        num_cores=NC, num_subcores=NS),
    scratch_types=[
        pltpu.VMEM((HALF + PAD,), jnp.int32),
        pltpu.VMEM((2 * G, 128), jnp.int32),
        pltpu.VMEM((2 * G, 128), jnp.int32),
        pltpu.VMEM((2, 128), jnp.int32),
        pltpu.VMEM((2, 128, D), jnp.float32),
        pltpu.VMEM((HALF, D), jnp.float32),
        pltpu.VMEM((HALF, D), jnp.float32),
        pltpu.SemaphoreType.DMA,
        pltpu.SemaphoreType.DMA,
        pltpu.SemaphoreType.DMA,
    ],
    compiler_params=pltpu.CompilerParams(needs_layout_passes=False),
)


def _tc_body(word_ref, hid_ref, p_ref, out_ref):
    acc = jnp.dot(hid_ref[...], p_ref[...],
                  preferred_element_type=jnp.float32,
                  precision=lax.Precision.HIGHEST)
    out_ref[...] = (word_ref[...] + acc) * np.float32(1.0 / 9.0)


_TB = 2048


def _tc_project(word_emb, hidden, pfull):
    return pl.pallas_call(
        _tc_body,
        grid=(T // _TB,),
        in_specs=[
            pl.BlockSpec((_TB, D), lambda i: (i, 0)),
            pl.BlockSpec((_TB, D), lambda i: (i, 0)),
            pl.BlockSpec((D, D), lambda i: (0, 0)),
        ],
        out_specs=pl.BlockSpec((_TB, D), lambda i: (i, 0)),
        out_shape=jax.ShapeDtypeStruct((T, D), jnp.float32),
    )(word_emb, hidden, pfull)


def kernel(input_ids, word_table, oe_table, oe_projection):
    with jax.enable_x64(False):
        toks32 = input_ids.astype(jnp.int32)
        padded = jnp.concatenate([jnp.zeros((PAD,), jnp.int32), toks32])
        oet = oe_table.T
        tail = jnp.pad(
            oet[:, TAIL2_C0:], ((0, 0), (0, 128 - (OE_ROWS - TAIL2_C0))))
        oe_lines = _sc_transpose(oet, tail)
        word_emb, hidden = _sc_gather(padded, word_table, oe_lines)
        pfull = oe_projection.reshape(D, D)
        return _tc_project(word_emb, hidden, pfull)


# speedup vs baseline: 1.0446x; 1.0446x over previous
import jax
import jax.numpy as jnp
import numpy as np
from jax import lax
from jax.experimental import pallas as pl
from jax.experimental.pallas import tpu as pltpu
from jax.experimental.pallas import tpu_sc as plsc

T = 16384
D = 128
H = 16
G = 8
NUM_EMB = 100000
OE_M = 100003
OE_ROWS = 800088
OE_LINES = 100016

NC = 2
NS = 16
NW = NC * NS
TPW = T // NW
HALF = TPW // 2
PAD = 8

CH = 2048
CHP = CH + 1
NCH_FULL = 390
NJ = 13
TAIL1_C0 = NCH_FULL * CH
TAIL1_W = 800000 - TAIL1_C0
TAIL2_C0 = 800000

_EXCL = [0] * (G + 1)
for _i in range(G):
    _EXCL[_i + 1] = _EXCL[_i] + OE_M + 2 * _i + 1


def _fmod(x, m, inv):
    m = np.int32(m)
    q = (x.astype(jnp.float32) * inv).astype(jnp.int32)
    r = x - q * m
    r = jnp.where(r < np.int32(0), r + m, r)
    r = jnp.where(r >= m, r - m, r)
    return r


def _tr_body(oet_hbm, tail_hbm, lines_out, chunk_v, out_v, isem0, isem1,
             osem):
    cid = lax.axis_index("c")
    sid = lax.axis_index("s")
    wid = sid * np.int32(NC) + cid
    iota16 = lax.iota(jnp.int32, 16)
    zeros16 = jnp.full((16,), 0, jnp.int32)
    scol = [iota16 + np.int32(16 * k) for k in range(8)]
    isems = [isem0, isem1]

    def in_copy2(jj, b):
        ch = wid + jj * np.int32(NW)
        return pltpu.make_async_copy(
            oet_hbm.at[:, pl.ds(ch * np.int32(CH), CH)],
            chunk_v.at[b, :, pl.ds(0, CH)], isems[b])

    def extract(src2d, L):
        @plsc.parallel_loop(np.int32(0), np.int32(L), np.int32(1), unroll=8)
        def lbody(l):
            base8 = l * np.int32(8)
            for k in range(8):
                col = zeros16 + (base8 + np.int32(k))
                vals = plsc.load_gather(src2d, [iota16, col])
                out_v[l, pl.ds(16 * k, 16)] = vals

    def out_copy(lbase, L):
        return pltpu.make_async_copy(
            out_v.at[pl.ds(0, L)], lines_out.at[pl.ds(lbase, L)], osem)

    @pl.when(wid < np.int32(NCH_FULL))
    def _():
        in_copy2(np.int32(0), 0).start()

    def j2body(j2, _):
        for b in range(2):
            jj = j2 * np.int32(2) + np.int32(b)
            ch = wid + jj * np.int32(NW)
            chn = ch + np.int32(NW)

            @pl.when(chn < np.int32(NCH_FULL))
            def _(jj=jj, b=b):
                in_copy2(jj + np.int32(1), 1 - b).start()

            @pl.when(ch < np.int32(NCH_FULL))
            def _(jj=jj, ch=ch, b=b):
                in_copy2(jj, b).wait()

                @pl.when(jj >= np.int32(1))
                def _():
                    out_copy(np.int32(0), 256).wait()

                extract(chunk_v.at[b], 256)
                out_copy(ch * np.int32(256), 256).start()

        return np.int32(0)

    lax.fori_loop(np.int32(0), np.int32((NJ + 1) // 2), j2body, np.int32(0))

    out_copy(np.int32(0), 256).wait()

    @pl.when(wid == np.int32(0))
    def _():
        pltpu.sync_copy(oet_hbm.at[:, pl.ds(TAIL1_C0, TAIL1_W)],
                        chunk_v.at[0, :, pl.ds(0, TAIL1_W)])
        extract(chunk_v.at[0], TAIL1_W // 8)
        out_copy(np.int32(TAIL1_C0 // 8), TAIL1_W // 8).start()
        out_copy(np.int32(0), TAIL1_W // 8).wait()

    @pl.when(wid == np.int32(1))
    def _():
        pltpu.sync_copy(tail_hbm, chunk_v.at[0, :, pl.ds(0, 128)])
        extract(chunk_v.at[0], 16)
        out_copy(np.int32(TAIL2_C0 // 8), 16).start()
        out_copy(np.int32(0), 16).wait()


_sc_transpose = pl.kernel(
    _tr_body,
    out_type=jax.ShapeDtypeStruct((OE_LINES, D), jnp.float32),
    mesh=plsc.VectorSubcoreMesh(
        core_axis_name="c", subcore_axis_name="s",
        num_cores=NC, num_subcores=NS),
    scratch_types=[
        pltpu.VMEM((2, 16, CHP), jnp.float32),
        pltpu.VMEM((256, D), jnp.float32),
        pltpu.SemaphoreType.DMA,
        pltpu.SemaphoreType.DMA,
        pltpu.SemaphoreType.DMA,
    ],
    compiler_params=pltpu.CompilerParams(
        needs_layout_passes=False, disable_bounds_checks=True),
)


def _sc_body(padded_hbm, wt_hbm, oel_hbm, word_out, hid_out,
             toks_v, lidx_v, soff_v, widx_v, lines_v, hid_v, wrows_v,
             lsem0, lsem1, wsem):
    cid = lax.axis_index("c")
    sid = lax.axis_index("s")
    wid = sid * np.int32(NC) + cid
    base = wid * np.int32(TPW)
    iota16 = lax.iota(jnp.int32, 16)
    rot = [(iota16 + np.int32(h)) & np.int32(15) for h in range(16)]
    lsems = [lsem0, lsem1]

    def line_copy(g, s, b):
        return pltpu.make_async_copy(
            oel_hbm.at[lidx_v.at[2 * g + s]], lines_v.at[b], lsems[b])

    def half_body(half, _):
        hbase = base + half * np.int32(HALF)
        pltpu.sync_copy(padded_hbm.at[pl.ds(hbase, HALF + PAD)], toks_v)

        for s in range(2):
            wrow = widx_v.at[s]

            def wbody(i, _, s=s, wrow=wrow):
                off = i * np.int32(16) + np.int32(s * 128 + PAD)
                wrow[pl.ds(i * np.int32(16), 16)] = toks_v[pl.ds(off, 16)]
                return np.int32(0)

            lax.fori_loop(np.int32(0), np.int32(8), wbody, np.int32(0))
        wdescs = [
            pltpu.make_async_copy(
                wt_hbm.at[widx_v.at[s]],
                wrows_v.at[pl.ds(s * 128, 128)], wsem)
            for s in range(2)
        ]
        for dsc in wdescs:
            dsc.start()

        for g in range(G):
            m = 100004 + 2 * g
            c = m - NUM_EMB
            inv = np.float32(1.0 / m)
            excl = _EXCL[g]
            for s in range(2):
                lrow = lidx_v.at[2 * g + s]
                orow = soff_v.at[2 * g + s]

                def gbody(i, _, s=s, m=m, c=c, inv=inv, excl=excl,
                          lrow=lrow, orow=orow, is3=(g >= 4)):
                    off = i * np.int32(16) + np.int32(s * 128 + PAD)
                    t0 = toks_v[pl.ds(off, 16)]
                    t1 = toks_v[pl.ds(off - np.int32(1), 16)]
                    r1 = _fmod(np.int32(c) * t1, m, inv)
                    acc = t0 - r1 + np.int32(m)
                    if is3:
                        t2 = toks_v[pl.ds(off - np.int32(2), 16)]
                        rr2 = _fmod(
                            np.int32(c) * _fmod(np.int32(c) * t2, m, inv),
                            m, inv)
                        acc = acc + rr2
                    gid = _fmod(acc, m, inv) + np.int32(excl)
                    j = i * np.int32(16)
                    lrow[pl.ds(j, 16)] = lax.shift_right_logical(
                        gid, np.int32(3))
                    orow[pl.ds(j, 16)] = lax.shift_left(
                        gid & np.int32(7), np.int32(4))
                    return np.int32(0)

                lax.fori_loop(np.int32(0), np.int32(8), gbody, np.int32(0))

        pairs = [(g, s) for g in range(G) for s in range(2)]
        line_copy(0, 0, 0).start()
        for p, (g, s) in enumerate(pairs):
            b = p % 2
            if p + 1 < len(pairs):
                gn, sn = pairs[p + 1]
                line_copy(gn, sn, 1 - b).start()
            line_copy(g, s, b).wait()
            src = lines_v.at[b]
            orow = soff_v.at[2 * g + s]

            @plsc.parallel_loop(np.int32(0), np.int32(8), np.int32(1),
                                unroll=4)
            def ebody(j, g=g, s=s, src=src, orow=orow):
                soff16 = orow[pl.ds(j * np.int32(16), 16)]
                lrows = iota16 + j * np.int32(16)
                trows = lrows + np.int32(s * 128)
                for h in range(16):
                    vals = plsc.load_gather(src, [lrows, soff16 + rot[h]])
                    plsc.store_scatter(
                        hid_v, [trows, rot[h] + np.int32(16 * g)], vals)

        for dsc in wdescs:
            dsc.wait()
        pltpu.sync_copy(wrows_v, word_out.at[pl.ds(hbase, HALF)])
        pltpu.sync_copy(hid_v, hid_out.at[pl.ds(hbase, HALF)])
        return np.int32(0)

    lax.fori_loop(np.int32(0), np.int32(2), half_body, np.int32(0))


_sc_gather = pl.kernel(
    _sc_body,
    out_type=(
        jax.ShapeDtypeStruct((T, D), jnp.float32),
        jax.ShapeDtypeStruct((T, D), jnp.float32),
    ),
    mesh=plsc.VectorSubcoreMesh(
        core_axis_name="c", subcore_axis_name="s",
        num_cores=NC, num_subcores=NS),
    scratch_types=[
        pltpu.VMEM((HALF + PAD,), jnp.int32),
        pltpu.VMEM((2 * G, 128), jnp.int32),
        pltpu.VMEM((2 * G, 128), jnp.int32),
        pltpu.VMEM((2, 128), jnp.int32),
        pltpu.VMEM((2, 128, D), jnp.float32),
        pltpu.VMEM((HALF, D), jnp.float32),
        pltpu.VMEM((HALF, D), jnp.float32),
        pltpu.SemaphoreType.DMA,
        pltpu.SemaphoreType.DMA,
        pltpu.SemaphoreType.DMA,
    ],
    compiler_params=pltpu.CompilerParams(
        needs_layout_passes=False, disable_bounds_checks=True),
)


def _tc_body(word_ref, hid_ref, p_ref, out_ref):
    acc = jnp.dot(hid_ref[...], p_ref[...],
                  preferred_element_type=jnp.float32,
                  precision=lax.Precision.HIGHEST)
    out_ref[...] = (word_ref[...] + acc) * np.float32(1.0 / 9.0)


_TB = 2048


def _tc_project(word_emb, hidden, pfull):
    return pl.pallas_call(
        _tc_body,
        grid=(T // _TB,),
        in_specs=[
            pl.BlockSpec((_TB, D), lambda i: (i, 0)),
            pl.BlockSpec((_TB, D), lambda i: (i, 0)),
            pl.BlockSpec((D, D), lambda i: (0, 0)),
        ],
        out_specs=pl.BlockSpec((_TB, D), lambda i: (i, 0)),
        out_shape=jax.ShapeDtypeStruct((T, D), jnp.float32),
    )(word_emb, hidden, pfull)


def kernel(input_ids, word_table, oe_table, oe_projection):
    with jax.enable_x64(False):
        toks32 = input_ids.astype(jnp.int32)
        padded = jnp.concatenate([jnp.zeros((PAD,), jnp.int32), toks32])
        oet = oe_table.T
        tail = jnp.pad(
            oet[:, TAIL2_C0:], ((0, 0), (0, 128 - (OE_ROWS - TAIL2_C0))))
        oe_lines = _sc_transpose(oet, tail)
        word_emb, hidden = _sc_gather(padded, word_table, oe_lines)
        pfull = oe_projection.reshape(D, D)
        return _tc_project(word_emb, hidden, pfull)


# speedup vs baseline: 1.0593x; 1.0141x over previous
import jax
import jax.numpy as jnp
import numpy as np
from jax import lax
from jax.experimental import pallas as pl
from jax.experimental.pallas import tpu as pltpu
from jax.experimental.pallas import tpu_sc as plsc

T = 16384
D = 128
H = 16
G = 8
NUM_EMB = 100000
OE_M = 100003
OE_ROWS = 800088
OE_LINES = 100016

NC = 2
NS = 16
NW = NC * NS
TPW = T // NW
HALF = TPW // 2
PAD = 8

CH = 2048
CHP = CH + 1
NCH_FULL = 390
NJ = 13
TAIL1_C0 = NCH_FULL * CH
TAIL1_W = 800000 - TAIL1_C0
TAIL2_C0 = 800000

_EXCL = [0] * (G + 1)
for _i in range(G):
    _EXCL[_i + 1] = _EXCL[_i] + OE_M + 2 * _i + 1


def _fmod(x, m, inv):
    m = np.int32(m)
    q = (x.astype(jnp.float32) * inv).astype(jnp.int32)
    r = x - q * m
    r = jnp.where(r < np.int32(0), r + m, r)
    r = jnp.where(r >= m, r - m, r)
    return r


def _tr_body(oet_hbm, tail_hbm, lines_out, chunk_v, out_v, isem0, isem1,
             osem):
    cid = lax.axis_index("c")
    sid = lax.axis_index("s")
    wid = sid * np.int32(NC) + cid
    iota16 = lax.iota(jnp.int32, 16)
    zeros16 = jnp.full((16,), 0, jnp.int32)
    scol = [iota16 + np.int32(16 * k) for k in range(8)]
    isems = [isem0, isem1]

    def in_copy2(jj, b):
        ch = wid + jj * np.int32(NW)
        return pltpu.make_async_copy(
            oet_hbm.at[:, pl.ds(ch * np.int32(CH), CH)],
            chunk_v.at[b, :, pl.ds(0, CH)], isems[b])

    def extract(src2d, L):
        @plsc.parallel_loop(np.int32(0), np.int32(L), np.int32(1), unroll=8)
        def lbody(l):
            base8 = l * np.int32(8)
            for k in range(8):
                col = zeros16 + (base8 + np.int32(k))
                vals = plsc.load_gather(src2d, [iota16, col])
                out_v[l, pl.ds(16 * k, 16)] = vals

    def out_copy(lbase, L):
        return pltpu.make_async_copy(
            out_v.at[pl.ds(0, L)], lines_out.at[pl.ds(lbase, L)], osem)

    @pl.when(wid < np.int32(NCH_FULL))
    def _():
        in_copy2(np.int32(0), 0).start()

    def j2body(j2, _):
        for b in range(2):
            jj = j2 * np.int32(2) + np.int32(b)
            ch = wid + jj * np.int32(NW)
            chn = ch + np.int32(NW)

            @pl.when(chn < np.int32(NCH_FULL))
            def _(jj=jj, b=b):
                in_copy2(jj + np.int32(1), 1 - b).start()

            @pl.when(ch < np.int32(NCH_FULL))
            def _(jj=jj, ch=ch, b=b):
                in_copy2(jj, b).wait()

                @pl.when(jj >= np.int32(1))
                def _():
                    out_copy(np.int32(0), 256).wait()

                extract(chunk_v.at[b], 256)
                out_copy(ch * np.int32(256), 256).start()

        return np.int32(0)

    lax.fori_loop(np.int32(0), np.int32((NJ + 1) // 2), j2body, np.int32(0))

    out_copy(np.int32(0), 256).wait()

    @pl.when(wid == np.int32(0))
    def _():
        pltpu.sync_copy(oet_hbm.at[:, pl.ds(TAIL1_C0, TAIL1_W)],
                        chunk_v.at[0, :, pl.ds(0, TAIL1_W)])
        extract(chunk_v.at[0], TAIL1_W // 8)
        out_copy(np.int32(TAIL1_C0 // 8), TAIL1_W // 8).start()
        out_copy(np.int32(0), TAIL1_W // 8).wait()

    @pl.when(wid == np.int32(1))
    def _():
        pltpu.sync_copy(tail_hbm, chunk_v.at[0, :, pl.ds(0, 128)])
        extract(chunk_v.at[0], 16)
        out_copy(np.int32(TAIL2_C0 // 8), 16).start()
        out_copy(np.int32(0), 16).wait()


_sc_transpose = pl.kernel(
    _tr_body,
    out_type=jax.ShapeDtypeStruct((OE_LINES, D), jnp.float32),
    mesh=plsc.VectorSubcoreMesh(
        core_axis_name="c", subcore_axis_name="s",
        num_cores=NC, num_subcores=NS),
    scratch_types=[
        pltpu.VMEM((2, 16, CHP), jnp.float32),
        pltpu.VMEM((256, D), jnp.float32),
        pltpu.SemaphoreType.DMA,
        pltpu.SemaphoreType.DMA,
        pltpu.SemaphoreType.DMA,
    ],
    compiler_params=pltpu.CompilerParams(
        needs_layout_passes=False, disable_bounds_checks=True),
)


def _sc_body(padded_hbm, wt_hbm, oel_hbm, word_out, hid_out,
             toks_v, lidx_v, soff_v, widx_v, lines_v, hid_v, wrows_v,
             lsem0, lsem1, lsem2, wsem):
    cid = lax.axis_index("c")
    sid = lax.axis_index("s")
    wid = sid * np.int32(NC) + cid
    base = wid * np.int32(TPW)
    iota16 = lax.iota(jnp.int32, 16)
    rot = [(iota16 + np.int32(h)) & np.int32(15) for h in range(16)]
    lsems = [lsem0, lsem1, lsem2]

    def line_copy(g, s, b):
        return pltpu.make_async_copy(
            oel_hbm.at[lidx_v.at[2 * g + s]], lines_v.at[b], lsems[b])

    def half_body(half, _):
        hbase = base + half * np.int32(HALF)
        pltpu.sync_copy(padded_hbm.at[pl.ds(hbase, HALF + PAD)], toks_v)

        for s in range(2):
            wrow = widx_v.at[s]

            def wbody(i, _, s=s, wrow=wrow):
                off = i * np.int32(16) + np.int32(s * 128 + PAD)
                wrow[pl.ds(i * np.int32(16), 16)] = toks_v[pl.ds(off, 16)]
                return np.int32(0)

            lax.fori_loop(np.int32(0), np.int32(8), wbody, np.int32(0))
        wdescs = [
            pltpu.make_async_copy(
                wt_hbm.at[widx_v.at[s]],
                wrows_v.at[pl.ds(s * 128, 128)], wsem)
            for s in range(2)
        ]
        for dsc in wdescs:
            dsc.start()

        for g in range(G):
            m = 100004 + 2 * g
            c = m - NUM_EMB
            inv = np.float32(1.0 / m)
            excl = _EXCL[g]
            for s in range(2):
                lrow = lidx_v.at[2 * g + s]
                orow = soff_v.at[2 * g + s]

                def gbody(i, _, s=s, m=m, c=c, inv=inv, excl=excl,
                          lrow=lrow, orow=orow, is3=(g >= 4)):
                    off = i * np.int32(16) + np.int32(s * 128 + PAD)
                    t0 = toks_v[pl.ds(off, 16)]
                    t1 = toks_v[pl.ds(off - np.int32(1), 16)]
                    r1 = _fmod(np.int32(c) * t1, m, inv)
                    acc = t0 - r1 + np.int32(m)
                    if is3:
                        t2 = toks_v[pl.ds(off - np.int32(2), 16)]
                        rr2 = _fmod(
                            np.int32(c) * _fmod(np.int32(c) * t2, m, inv),
                            m, inv)
                        acc = acc + rr2
                    gid = _fmod(acc, m, inv) + np.int32(excl)
                    j = i * np.int32(16)
                    lrow[pl.ds(j, 16)] = lax.shift_right_logical(
                        gid, np.int32(3))
                    orow[pl.ds(j, 16)] = lax.shift_left(
                        gid & np.int32(7), np.int32(4))
                    return np.int32(0)

                lax.fori_loop(np.int32(0), np.int32(8), gbody, np.int32(0))

        pairs = [(g, s) for g in range(G) for s in range(2)]
        line_copy(*pairs[0], 0).start()
        line_copy(*pairs[1], 1).start()
        for p, (g, s) in enumerate(pairs):
            b = p % 3
            if p + 2 < len(pairs):
                gn, sn = pairs[p + 2]
                line_copy(gn, sn, (p + 2) % 3).start()
            line_copy(g, s, b).wait()
            src = lines_v.at[b]
            orow = soff_v.at[2 * g + s]

            @plsc.parallel_loop(np.int32(0), np.int32(8), np.int32(1),
                                unroll=4)
            def ebody(j, g=g, s=s, src=src, orow=orow):
                soff16 = orow[pl.ds(j * np.int32(16), 16)]
                lrows = iota16 + j * np.int32(16)
                trows = lrows + np.int32(s * 128)
                for h in range(16):
                    vals = plsc.load_gather(src, [lrows, soff16 + rot[h]])
                    plsc.store_scatter(
                        hid_v, [trows, rot[h] + np.int32(16 * g)], vals)

        for dsc in wdescs:
            dsc.wait()
        pltpu.sync_copy(wrows_v, word_out.at[pl.ds(hbase, HALF)])
        pltpu.sync_copy(hid_v, hid_out.at[pl.ds(hbase, HALF)])
        return np.int32(0)

    lax.fori_loop(np.int32(0), np.int32(2), half_body, np.int32(0))


_sc_gather = pl.kernel(
    _sc_body,
    out_type=(
        jax.ShapeDtypeStruct((T, D), jnp.float32),
        jax.ShapeDtypeStruct((T, D), jnp.float32),
    ),
    mesh=plsc.VectorSubcoreMesh(
        core_axis_name="c", subcore_axis_name="s",
        num_cores=NC, num_subcores=NS),
    scratch_types=[
        pltpu.VMEM((HALF + PAD,), jnp.int32),
        pltpu.VMEM((2 * G, 128), jnp.int32),
        pltpu.VMEM((2 * G, 128), jnp.int32),
        pltpu.VMEM((2, 128), jnp.int32),
        pltpu.VMEM((3, 128, D), jnp.float32),
        pltpu.VMEM((HALF, D), jnp.float32),
        pltpu.VMEM((HALF, D), jnp.float32),
        pltpu.SemaphoreType.DMA,
        pltpu.SemaphoreType.DMA,
        pltpu.SemaphoreType.DMA,
        pltpu.SemaphoreType.DMA,
    ],
    compiler_params=pltpu.CompilerParams(
        needs_layout_passes=False, disable_bounds_checks=True),
)


def _tc_body(word_ref, hid_ref, p_ref, out_ref):
    acc = jnp.dot(hid_ref[...], p_ref[...],
                  preferred_element_type=jnp.float32,
                  precision=lax.Precision.HIGHEST)
    out_ref[...] = (word_ref[...] + acc) * np.float32(1.0 / 9.0)


_TB = 2048


def _tc_project(word_emb, hidden, pfull):
    return pl.pallas_call(
        _tc_body,
        grid=(T // _TB,),
        in_specs=[
            pl.BlockSpec((_TB, D), lambda i: (i, 0)),
            pl.BlockSpec((_TB, D), lambda i: (i, 0)),
            pl.BlockSpec((D, D), lambda i: (0, 0)),
        ],
        out_specs=pl.BlockSpec((_TB, D), lambda i: (i, 0)),
        out_shape=jax.ShapeDtypeStruct((T, D), jnp.float32),
    )(word_emb, hidden, pfull)


def kernel(input_ids, word_table, oe_table, oe_projection):
    with jax.enable_x64(False):
        toks32 = input_ids.astype(jnp.int32)
        padded = jnp.concatenate([jnp.zeros((PAD,), jnp.int32), toks32])
        oet = oe_table.T
        tail = jnp.pad(
            oet[:, TAIL2_C0:], ((0, 0), (0, 128 - (OE_ROWS - TAIL2_C0))))
        oe_lines = _sc_transpose(oet, tail)
        word_emb, hidden = _sc_gather(padded, word_table, oe_lines)
        pfull = oe_projection.reshape(D, D)
        return _tc_project(word_emb, hidden, pfull)
